# Initial kernel scaffold; baseline (speedup 1.0000x reference)
import jax
import jax.numpy as jnp
from jax.experimental import pallas as pl


def kernel(z_e, codebook):
    raise NotImplementedError("write your pallas kernel here")


# speedup vs baseline: 1.5755x; 1.5755x over previous
import functools

import jax
import jax.numpy as jnp
from jax import lax
from jax.experimental import pallas as pl
from jax.experimental.pallas import tpu as pltpu
from jax.experimental.pallas import tpu_sc as plsc

K = 1024
D = 64
BN = 1024

NC = 2
NS = 16
NW = NC * NS
CHUNK = 128


def _tc_body(x_ref, cbt_ref, idx_ref, loss_ref, acc_ref):
    i = pl.program_id(0)
    nsteps = pl.num_programs(0)
    x = x_ref[:, :]
    cbt = cbt_ref[:, :]
    e2 = jnp.sum(cbt * cbt, axis=0, keepdims=True)
    x2 = jnp.sum(x * x, axis=1, keepdims=True)
    dot = lax.dot_general(x, cbt, (((1,), (0,)), ((), ())),
                          preferred_element_type=jnp.float32)
    d2 = x2 + e2 - 2.0 * dot
    d2c = jnp.maximum(d2, 0.0)
    minv = jnp.min(d2c, axis=1, keepdims=True)
    iota = lax.broadcasted_iota(jnp.int32, (BN, K), 1)
    idx_ref[:] = jnp.min(jnp.where(d2c == minv, iota, K), axis=1)

    @pl.when(i == 0)
    def _init():
        acc_ref[0] = 0.0

    acc_ref[0] += jnp.sum(minv)

    @pl.when(i == nsteps - 1)
    def _fin():
        n_total = nsteps * BN
        loss_ref[0, 0] = acc_ref[0] * (1.25 / (n_total * D))


def _tc_distance_argmin(flat, codebook):
    n = flat.shape[0]
    grid = (n // BN,)
    return pl.pallas_call(
        _tc_body,
        grid=grid,
        in_specs=[
            pl.BlockSpec((BN, D), lambda i: (i, 0)),
            pl.BlockSpec((D, K), lambda i: (0, 0)),
        ],
        out_specs=[
            pl.BlockSpec((BN,), lambda i: (i,)),
            pl.BlockSpec(memory_space=pltpu.SMEM),
        ],
        out_shape=[
            jax.ShapeDtypeStruct((n,), jnp.int32),
            jax.ShapeDtypeStruct((1, 1), jnp.float32),
        ],
        scratch_shapes=[pltpu.SMEM((1,), jnp.float32)],
    )(flat, codebook.T)


PD = 128


def _make_sc_gather(n_rows):
    n_chunks = n_rows // CHUNK
    rows_per_w = n_chunks // NW
    mesh = plsc.VectorSubcoreMesh(core_axis_name="c", subcore_axis_name="s")

    @functools.partial(
        pl.kernel,
        mesh=mesh,
        out_type=jax.ShapeDtypeStruct((n_chunks, CHUNK, PD), jnp.float32),
        scratch_types=[
            pltpu.VMEM((rows_per_w, CHUNK), jnp.int32),
            pltpu.VMEM((2, CHUNK, PD), jnp.float32),
            pltpu.SemaphoreType.DMA,
        ],
    )
    def sc_gather(cb_hbm, idx_hbm, out_hbm, idx_v, rows_v, sem):
        wid = lax.axis_index("s") * NC + lax.axis_index("c")
        base = wid * rows_per_w
        pltpu.sync_copy(idx_hbm.at[wid], idx_v)
        copies = [None] * rows_per_w
        copies[0] = pltpu.async_copy(cb_hbm.at[idx_v.at[0]], rows_v.at[0], sem)
        for j in range(rows_per_w):
            if j + 1 < rows_per_w:
                copies[j + 1] = pltpu.async_copy(
                    cb_hbm.at[idx_v.at[j + 1]], rows_v.at[(j + 1) % 2], sem)
            copies[j].wait()
            pltpu.sync_copy(rows_v.at[j % 2], out_hbm.at[base + j])

    return sc_gather


def kernel(z_e, codebook):
    shape = z_e.shape
    flat = z_e.reshape(-1, D)
    n = flat.shape[0]
    idx, loss = _tc_distance_argmin(flat, codebook)
    cb_pad = jnp.pad(codebook, ((0, 0), (0, PD - D)))
    z_q = _make_sc_gather(n)(cb_pad, idx.reshape(NW, -1, CHUNK))
    return (z_q[..., :D].reshape(shape), loss[0, 0], idx.reshape(shape[:-1]))
